# baseline (speedup 1.0000x reference)
import functools

import jax
import jax.numpy as jnp
from jax.experimental import pallas as pl
from jax.experimental.pallas import tpu as pltpu
from jax.experimental.pallas import tpu_sc as plsc

CLS = (30, 100, 300, 1000, 3000, 8000)
BATCH_BLK = 256
HEAD_BLK = 256
CHAIN_BLK = 256


def _sc_row_partials(mat):
    rows, cols = mat.shape
    lanes = 16
    nworkers = 32
    nway = 4
    ngroups = rows // lanes
    nfull = cols // lanes
    npair = nfull // nway
    rem = nfull - npair * nway
    max_t = -(-ngroups // nworkers)
    mesh = plsc.VectorSubcoreMesh(core_axis_name="c", subcore_axis_name="s")

    @functools.partial(
        pl.kernel,
        mesh=mesh,
        out_type=(jax.ShapeDtypeStruct((ngroups, lanes * lanes), jnp.float32),
                  jax.ShapeDtypeStruct((ngroups, lanes * lanes), jnp.int32)),
        scratch_types=[
            pltpu.VMEM((lanes, cols), jnp.float32),
            pltpu.VMEM((lanes, cols), jnp.float32),
            pltpu.VMEM((lanes * lanes,), jnp.float32),
            pltpu.VMEM((lanes * lanes,), jnp.int32),
            pltpu.SemaphoreType.DMA,
            pltpu.SemaphoreType.DMA,
        ],
    )
    def k(m_hbm, oval_hbm, oidx_hbm, buf0, buf1, sval, sidx, sem0, sem1):
        cid = jax.lax.axis_index("c")
        sid = jax.lax.axis_index("s")
        wid = sid * 2 + cid
        bufs = (buf0, buf1)
        sems = (sem0, sem1)
        lane = jax.lax.iota(jnp.int32, lanes)
        neg = jnp.full((lanes,), -jnp.inf, jnp.float32)

        def startg(g, slot):
            @pl.when(g < ngroups)
            def _():
                pltpu.async_copy(m_hbm.at[pl.ds(g * lanes, lanes), :],
                                 bufs[slot], sems[slot])

        def compute(g, buf):
            for i in range(lanes):
                def body(j, carry):
                    accs, colv = carry
                    new = []
                    for w, (amax, aidx) in enumerate(accs):
                        v = buf[i, pl.ds((j * nway + w) * lanes, lanes)]
                        cw = colv + w * lanes
                        upd = v > amax
                        new.append((jnp.where(upd, v, amax),
                                    jnp.where(upd, cw, aidx)))
                    return (tuple(new), colv + nway * lanes)

                init = (tuple((neg, jnp.zeros((lanes,), jnp.int32))
                              for _ in range(nway)), lane)
                accs, colv = jax.lax.fori_loop(0, npair, body, init)
                cmax, cidx = accs[0]
                for amax, aidx in accs[1:]:
                    better = (amax > cmax) | ((amax == cmax) & (aidx < cidx))
                    cmax = jnp.where(better, amax, cmax)
                    cidx = jnp.where(better, aidx, cidx)
                for w in range(rem):
                    v = buf[i, pl.ds((npair * nway + w) * lanes, lanes)]
                    cw = colv + w * lanes
                    upd = v > cmax
                    cmax = jnp.where(upd, v, cmax)
                    cidx = jnp.where(upd, cw, cidx)
                v = buf[i, pl.ds(cols - lanes, lanes)]
                v = jnp.where(lane >= nfull * lanes - (cols - lanes), v, neg)
                upd = v > cmax
                cmax = jnp.where(upd, v, cmax)
                cidx = jnp.where(upd, cols - lanes + lane, cidx)
                sval[pl.ds(i * lanes, lanes)] = cmax
                sidx[pl.ds(i * lanes, lanes)] = cidx
            pltpu.sync_copy(sval, oval_hbm.at[g])
            pltpu.sync_copy(sidx, oidx_hbm.at[g])

        def outer(tt, _):
            for slot in (0, 1):
                g = wid + nworkers * (2 * tt + slot)

                @pl.when(g < ngroups)
                def _(g=g, slot=slot):
                    pltpu.make_async_copy(m_hbm.at[pl.ds(g * lanes, lanes), :],
                                          bufs[slot], sems[slot]).wait()
                    compute(g, bufs[slot])
                    startg(g + 2 * nworkers, slot)
            return 0

        startg(wid, 0)
        startg(wid + nworkers, 1)
        jax.lax.fori_loop(0, (max_t + 1) // 2, outer, 0)

    return k(mat)


def _tables_kernel(val_ref, idx_ref, m3_ref, m2_ref, m1_ref, m0_ref,
                   p4_ref, p3_ref, p2_ref, p1_ref, p0_ref):
    val = val_ref[...]
    idx = idx_ref[...]
    m = jnp.max(val, axis=1, keepdims=True)
    p4_ref[...] = jnp.min(jnp.where(val == m, idx, jnp.int32(2**30)),
                          axis=1, keepdims=True)
    p3_ref[...] = _first_argmax(m3_ref[...])
    p2_ref[...] = _first_argmax(m2_ref[...])
    p1_ref[...] = _first_argmax(m1_ref[...])
    p0_ref[...] = _first_argmax(m0_ref[...])


def _parent_tables(m4, m3, m2, m1, m0):
    lanes = 16
    oval, oidx = _sc_row_partials(m4)
    val = oval.reshape(CLS[5], lanes)
    idx = oidx.reshape(CLS[5], lanes)
    full = lambda shape: pl.BlockSpec(shape, lambda: (0,) * len(shape))
    return pl.pallas_call(
        _tables_kernel,
        in_specs=[
            full((CLS[5], lanes)),
            full((CLS[5], lanes)),
            full(m3.shape),
            full(m2.shape),
            full(m1.shape),
            full(m0.shape),
        ],
        out_specs=[
            full((CLS[5], 1)),
            full((CLS[4], 1)),
            full((CLS[3], 1)),
            full((CLS[2], 1)),
            full((CLS[1], 1)),
        ],
        out_shape=[
            jax.ShapeDtypeStruct((CLS[5], 1), jnp.int32),
            jax.ShapeDtypeStruct((CLS[4], 1), jnp.int32),
            jax.ShapeDtypeStruct((CLS[3], 1), jnp.int32),
            jax.ShapeDtypeStruct((CLS[2], 1), jnp.int32),
            jax.ShapeDtypeStruct((CLS[1], 1), jnp.int32),
        ],
    )(val, idx, m3, m2, m1, m0)


def _first_argmax(vals):
    m = jnp.max(vals, axis=1, keepdims=True)
    iota = jax.lax.broadcasted_iota(jnp.int32, vals.shape, 1)
    return jnp.min(jnp.where(vals == m, iota, jnp.int32(2**30)),
                   axis=1, keepdims=True)


def _head_kernel(x_ref, w_ref, b_ref, sm_ref, pred_ref):
    logits = jnp.dot(x_ref[...], w_ref[...],
                     preferred_element_type=jnp.float32) + b_ref[...]
    m = jnp.max(logits, axis=1, keepdims=True)
    e = jnp.exp(logits - m)
    s = jnp.sum(e, axis=1, keepdims=True)
    sm_ref[...] = e / s
    pred_ref[...] = _first_argmax(logits)


def _rowargmax_kernel(m_ref, out_ref):
    out_ref[...] = _first_argmax(m_ref[...])


def _row_argmax(mat, row_blk):
    rows, cols = mat.shape
    grid = rows // row_blk
    return pl.pallas_call(
        _rowargmax_kernel,
        grid=(grid,),
        in_specs=[pl.BlockSpec((row_blk, cols), lambda i: (i, 0))],
        out_specs=pl.BlockSpec((row_blk, 1), lambda i: (i, 0)),
        out_shape=jax.ShapeDtypeStruct((rows, 1), jnp.int32),
    )(mat)


def _mega_kernel(x_ref, w_ref, b_ref, p4_ref, p3_ref, p2_ref, p1_ref, p0_ref,
                 sm_ref, o4_ref, o3_ref, o2_ref, o1_ref, o0_ref):
    logits = jnp.dot(x_ref[...], w_ref[...],
                     preferred_element_type=jnp.float32) + b_ref[...]
    m = jnp.max(logits, axis=1, keepdims=True)
    e = jnp.exp(logits - m)
    s = jnp.sum(e, axis=1, keepdims=True)
    sm_ref[...] = e / s
    pred = _first_argmax(logits)
    bsz = pred.shape[0]
    steps = ((p4_ref, o4_ref, CLS[5], CLS[4]),
             (p3_ref, o3_ref, CLS[4], CLS[3]),
             (p2_ref, o2_ref, CLS[3], CLS[2]),
             (p1_ref, o1_ref, CLS[2], CLS[1]),
             (p0_ref, o0_ref, CLS[1], CLS[0]))
    for t_ref, o_ref, dom, rng in steps:
        iota = jax.lax.broadcasted_iota(jnp.int32, (bsz, dom), 1)
        mask = pred == iota
        pred = jnp.sum(jnp.where(mask, t_ref[...], 0), axis=1, keepdims=True)
        iota2 = jax.lax.broadcasted_iota(jnp.int32, (bsz, rng), 1)
        o_ref[...] = (pred == iota2).astype(jnp.float32)


def kernel(x, W, b, m0, m1, m2, m3, m4):
    n = x.shape[0]
    d_in = x.shape[1]
    grid = n // BATCH_BLK

    p4, p3, p2, p1, p0 = _parent_tables(m4, m3, m2, m1, m0)
    tables = (p4.reshape(1, CLS[5]), p3.reshape(1, CLS[4]),
              p2.reshape(1, CLS[3]), p1.reshape(1, CLS[2]),
              p0.reshape(1, CLS[1]))

    outs = pl.pallas_call(
        _mega_kernel,
        grid=(grid,),
        in_specs=[
            pl.BlockSpec((BATCH_BLK, d_in), lambda i: (i, 0)),
            pl.BlockSpec((d_in, CLS[5]), lambda i: (0, 0)),
            pl.BlockSpec((1, CLS[5]), lambda i: (0, 0)),
            pl.BlockSpec((1, CLS[5]), lambda i: (0, 0)),
            pl.BlockSpec((1, CLS[4]), lambda i: (0, 0)),
            pl.BlockSpec((1, CLS[3]), lambda i: (0, 0)),
            pl.BlockSpec((1, CLS[2]), lambda i: (0, 0)),
            pl.BlockSpec((1, CLS[1]), lambda i: (0, 0)),
        ],
        out_specs=[
            pl.BlockSpec((BATCH_BLK, CLS[5]), lambda i: (i, 0)),
            pl.BlockSpec((BATCH_BLK, CLS[4]), lambda i: (i, 0)),
            pl.BlockSpec((BATCH_BLK, CLS[3]), lambda i: (i, 0)),
            pl.BlockSpec((BATCH_BLK, CLS[2]), lambda i: (i, 0)),
            pl.BlockSpec((BATCH_BLK, CLS[1]), lambda i: (i, 0)),
            pl.BlockSpec((BATCH_BLK, CLS[0]), lambda i: (i, 0)),
        ],
        out_shape=[
            jax.ShapeDtypeStruct((n, CLS[5]), jnp.float32),
            jax.ShapeDtypeStruct((n, CLS[4]), jnp.float32),
            jax.ShapeDtypeStruct((n, CLS[3]), jnp.float32),
            jax.ShapeDtypeStruct((n, CLS[2]), jnp.float32),
            jax.ShapeDtypeStruct((n, CLS[1]), jnp.float32),
            jax.ShapeDtypeStruct((n, CLS[0]), jnp.float32),
        ],
        compiler_params=pltpu.CompilerParams(
            vmem_limit_bytes=100 * 1024 * 1024),
    )(x, W, b.reshape(1, CLS[5]), *tables)

    sm, o4, o3, o2, o1, o0 = outs
    return (o0, o1, o2, o3, o4, sm)


# speedup vs baseline: 1.0671x; 1.0671x over previous
import functools

import jax
import jax.numpy as jnp
from jax.experimental import pallas as pl
from jax.experimental.pallas import tpu as pltpu
from jax.experimental.pallas import tpu_sc as plsc

CLS = (30, 100, 300, 1000, 3000, 8000)
BATCH_BLK = 256
HEAD_BLK = 256
CHAIN_BLK = 256


def _sc_row_partials(mat):
    rows, cols = mat.shape
    lanes = 16
    nworkers = 32
    nway = 4
    ngroups = rows // lanes
    nfull = cols // lanes
    npair = nfull // nway
    rem = nfull - npair * nway
    max_t = -(-ngroups // nworkers)
    mesh = plsc.VectorSubcoreMesh(core_axis_name="c", subcore_axis_name="s")

    @functools.partial(
        pl.kernel,
        mesh=mesh,
        out_type=(jax.ShapeDtypeStruct((ngroups, lanes * lanes), jnp.float32),
                  jax.ShapeDtypeStruct((ngroups, lanes * lanes), jnp.int32)),
        scratch_types=[
            pltpu.VMEM((lanes, cols), jnp.float32),
            pltpu.VMEM((lanes, cols), jnp.float32),
            pltpu.VMEM((lanes * lanes,), jnp.float32),
            pltpu.VMEM((lanes * lanes,), jnp.int32),
            pltpu.SemaphoreType.DMA,
            pltpu.SemaphoreType.DMA,
        ],
    )
    def k(m_hbm, oval_hbm, oidx_hbm, buf0, buf1, sval, sidx, sem0, sem1):
        cid = jax.lax.axis_index("c")
        sid = jax.lax.axis_index("s")
        wid = sid * 2 + cid
        bufs = (buf0, buf1)
        sems = (sem0, sem1)
        lane = jax.lax.iota(jnp.int32, lanes)
        neg = jnp.full((lanes,), -jnp.inf, jnp.float32)

        def startg(g, slot):
            @pl.when(g < ngroups)
            def _():
                pltpu.async_copy(m_hbm.at[pl.ds(g * lanes, lanes), :],
                                 bufs[slot], sems[slot])

        def compute(g, buf):
            for i in range(lanes):
                def body(j, carry):
                    accs, colv = carry
                    new = []
                    for w, (amax, aidx) in enumerate(accs):
                        v = buf[i, pl.ds((j * nway + w) * lanes, lanes)]
                        cw = colv + w * lanes
                        upd = v > amax
                        new.append((jnp.where(upd, v, amax),
                                    jnp.where(upd, cw, aidx)))
                    return (tuple(new), colv + nway * lanes)

                init = (tuple((neg, jnp.zeros((lanes,), jnp.int32))
                              for _ in range(nway)), lane)
                accs, colv = jax.lax.fori_loop(0, npair, body, init)
                cmax, cidx = accs[0]
                for amax, aidx in accs[1:]:
                    better = (amax > cmax) | ((amax == cmax) & (aidx < cidx))
                    cmax = jnp.where(better, amax, cmax)
                    cidx = jnp.where(better, aidx, cidx)
                for w in range(rem):
                    v = buf[i, pl.ds((npair * nway + w) * lanes, lanes)]
                    cw = colv + w * lanes
                    upd = v > cmax
                    cmax = jnp.where(upd, v, cmax)
                    cidx = jnp.where(upd, cw, cidx)
                v = buf[i, pl.ds(cols - lanes, lanes)]
                v = jnp.where(lane >= nfull * lanes - (cols - lanes), v, neg)
                upd = v > cmax
                cmax = jnp.where(upd, v, cmax)
                cidx = jnp.where(upd, cols - lanes + lane, cidx)
                sval[pl.ds(i * lanes, lanes)] = cmax
                sidx[pl.ds(i * lanes, lanes)] = cidx
            pltpu.sync_copy(sval, oval_hbm.at[g])
            pltpu.sync_copy(sidx, oidx_hbm.at[g])

        def outer(tt, _):
            for slot in (0, 1):
                g = wid + nworkers * (2 * tt + slot)

                @pl.when(g < ngroups)
                def _(g=g, slot=slot):
                    pltpu.make_async_copy(m_hbm.at[pl.ds(g * lanes, lanes), :],
                                          bufs[slot], sems[slot]).wait()
                    compute(g, bufs[slot])
                    startg(g + 2 * nworkers, slot)
            return 0

        startg(wid, 0)
        startg(wid + nworkers, 1)
        jax.lax.fori_loop(0, (max_t + 1) // 2, outer, 0)

    return k(mat)


def _tables_kernel(val_ref, idx_ref, m3_ref, m2_ref, m1_ref, m0_ref,
                   p4_ref, p3_ref, p2_ref, p1_ref, p0_ref):
    val = val_ref[...]
    idx = idx_ref[...]
    m = jnp.max(val, axis=1, keepdims=True)
    p4_ref[...] = jnp.min(jnp.where(val == m, idx, jnp.int32(2**30)),
                          axis=1, keepdims=True)
    p3_ref[...] = _first_argmax(m3_ref[...])
    p2_ref[...] = _first_argmax(m2_ref[...])
    p1_ref[...] = _first_argmax(m1_ref[...])
    p0_ref[...] = _first_argmax(m0_ref[...])


def _parent_tables(m4, m3, m2, m1, m0):
    lanes = 16
    oval, oidx = _sc_row_partials(m4)
    val = oval.reshape(CLS[5], lanes)
    idx = oidx.reshape(CLS[5], lanes)
    full = lambda shape: pl.BlockSpec(shape, lambda: (0,) * len(shape))
    return pl.pallas_call(
        _tables_kernel,
        in_specs=[
            full((CLS[5], lanes)),
            full((CLS[5], lanes)),
            full(m3.shape),
            full(m2.shape),
            full(m1.shape),
            full(m0.shape),
        ],
        out_specs=[
            full((CLS[5], 1)),
            full((CLS[4], 1)),
            full((CLS[3], 1)),
            full((CLS[2], 1)),
            full((CLS[1], 1)),
        ],
        out_shape=[
            jax.ShapeDtypeStruct((CLS[5], 1), jnp.int32),
            jax.ShapeDtypeStruct((CLS[4], 1), jnp.int32),
            jax.ShapeDtypeStruct((CLS[3], 1), jnp.int32),
            jax.ShapeDtypeStruct((CLS[2], 1), jnp.int32),
            jax.ShapeDtypeStruct((CLS[1], 1), jnp.int32),
        ],
    )(val, idx, m3, m2, m1, m0)


def _first_argmax(vals):
    m = jnp.max(vals, axis=1, keepdims=True)
    iota = jax.lax.broadcasted_iota(jnp.int32, vals.shape, 1)
    return jnp.min(jnp.where(vals == m, iota, jnp.int32(2**30)),
                   axis=1, keepdims=True)


def _head_kernel(x_ref, w_ref, b_ref, sm_ref, pred_ref):
    logits = jnp.dot(x_ref[...], w_ref[...],
                     preferred_element_type=jnp.float32) + b_ref[...]
    m = jnp.max(logits, axis=1, keepdims=True)
    e = jnp.exp(logits - m)
    s = jnp.sum(e, axis=1, keepdims=True)
    sm = e / s
    sm_ref[...] = sm
    pred_ref[...] = _first_argmax(sm)


def _rowargmax_kernel(m_ref, out_ref):
    out_ref[...] = _first_argmax(m_ref[...])


def _row_argmax(mat, row_blk):
    rows, cols = mat.shape
    grid = rows // row_blk
    return pl.pallas_call(
        _rowargmax_kernel,
        grid=(grid,),
        in_specs=[pl.BlockSpec((row_blk, cols), lambda i: (i, 0))],
        out_specs=pl.BlockSpec((row_blk, 1), lambda i: (i, 0)),
        out_shape=jax.ShapeDtypeStruct((rows, 1), jnp.int32),
    )(mat)


def _chain_kernel(pred5_ref, p4_ref, p3_ref, p2_ref, p1_ref, p0_ref,
                  o4_ref, o3_ref, o2_ref, o1_ref, o0_ref):
    pred = pred5_ref[...]
    bsz = pred.shape[0]
    steps = ((p4_ref, o4_ref, CLS[5], CLS[4]),
             (p3_ref, o3_ref, CLS[4], CLS[3]),
             (p2_ref, o2_ref, CLS[3], CLS[2]),
             (p1_ref, o1_ref, CLS[2], CLS[1]),
             (p0_ref, o0_ref, CLS[1], CLS[0]))
    for t_ref, o_ref, dom, rng in steps:
        iota = jax.lax.broadcasted_iota(jnp.int32, (bsz, dom), 1)
        mask = pred == iota
        pred = jnp.sum(jnp.where(mask, t_ref[...], 0), axis=1, keepdims=True)
        iota2 = jax.lax.broadcasted_iota(jnp.int32, (bsz, rng), 1)
        o_ref[...] = (pred == iota2).astype(jnp.float32)


def kernel(x, W, b, m0, m1, m2, m3, m4):
    n = x.shape[0]
    d_in = x.shape[1]
    grid = n // BATCH_BLK

    p4, p3, p2, p1, p0 = _parent_tables(m4, m3, m2, m1, m0)
    tables = (p4.reshape(1, CLS[5]), p3.reshape(1, CLS[4]),
              p2.reshape(1, CLS[3]), p1.reshape(1, CLS[2]),
              p0.reshape(1, CLS[1]))

    sm, pred5 = pl.pallas_call(
        _head_kernel,
        grid=(grid,),
        in_specs=[
            pl.BlockSpec((BATCH_BLK, d_in), lambda i: (i, 0)),
            pl.BlockSpec((d_in, CLS[5]), lambda i: (0, 0)),
            pl.BlockSpec((1, CLS[5]), lambda i: (0, 0)),
        ],
        out_specs=[
            pl.BlockSpec((BATCH_BLK, CLS[5]), lambda i: (i, 0)),
            pl.BlockSpec((BATCH_BLK, 1), lambda i: (i, 0)),
        ],
        out_shape=[
            jax.ShapeDtypeStruct((n, CLS[5]), jnp.float32),
            jax.ShapeDtypeStruct((n, 1), jnp.int32),
        ],
        compiler_params=pltpu.CompilerParams(
            vmem_limit_bytes=100 * 1024 * 1024),
    )(x, W, b.reshape(1, CLS[5]))

    o4, o3, o2, o1, o0 = pl.pallas_call(
        _chain_kernel,
        grid=(grid,),
        in_specs=[
            pl.BlockSpec((BATCH_BLK, 1), lambda i: (i, 0)),
            pl.BlockSpec((1, CLS[5]), lambda i: (0, 0)),
            pl.BlockSpec((1, CLS[4]), lambda i: (0, 0)),
            pl.BlockSpec((1, CLS[3]), lambda i: (0, 0)),
            pl.BlockSpec((1, CLS[2]), lambda i: (0, 0)),
            pl.BlockSpec((1, CLS[1]), lambda i: (0, 0)),
        ],
        out_specs=[
            pl.BlockSpec((BATCH_BLK, CLS[4]), lambda i: (i, 0)),
            pl.BlockSpec((BATCH_BLK, CLS[3]), lambda i: (i, 0)),
            pl.BlockSpec((BATCH_BLK, CLS[2]), lambda i: (i, 0)),
            pl.BlockSpec((BATCH_BLK, CLS[1]), lambda i: (i, 0)),
            pl.BlockSpec((BATCH_BLK, CLS[0]), lambda i: (i, 0)),
        ],
        out_shape=[
            jax.ShapeDtypeStruct((n, CLS[4]), jnp.float32),
            jax.ShapeDtypeStruct((n, CLS[3]), jnp.float32),
            jax.ShapeDtypeStruct((n, CLS[2]), jnp.float32),
            jax.ShapeDtypeStruct((n, CLS[1]), jnp.float32),
            jax.ShapeDtypeStruct((n, CLS[0]), jnp.float32),
        ],
        compiler_params=pltpu.CompilerParams(
            vmem_limit_bytes=100 * 1024 * 1024),
    )(pred5, *tables)

    return (o0, o1, o2, o3, o4, sm)


# speedup vs baseline: 1.0675x; 1.0004x over previous
import functools

import jax
import jax.numpy as jnp
from jax.experimental import pallas as pl
from jax.experimental.pallas import tpu as pltpu
from jax.experimental.pallas import tpu_sc as plsc

CLS = (30, 100, 300, 1000, 3000, 8000)
BATCH_BLK = 256


def _sc_row_partials(mat):
    rows, cols = mat.shape
    lanes = 16
    nworkers = 32
    nway = 4
    ngroups = rows // lanes
    nfull = cols // lanes
    npair = nfull // nway
    rem = nfull - npair * nway
    max_t = -(-ngroups // nworkers)
    mesh = plsc.VectorSubcoreMesh(core_axis_name="c", subcore_axis_name="s")

    @functools.partial(
        pl.kernel,
        mesh=mesh,
        out_type=(jax.ShapeDtypeStruct((ngroups, lanes * lanes), jnp.float32),
                  jax.ShapeDtypeStruct((ngroups, lanes * lanes), jnp.int32)),
        scratch_types=[
            pltpu.VMEM((lanes, cols), jnp.float32),
            pltpu.VMEM((lanes, cols), jnp.float32),
            pltpu.VMEM((lanes * lanes,), jnp.float32),
            pltpu.VMEM((lanes * lanes,), jnp.int32),
            pltpu.SemaphoreType.DMA,
            pltpu.SemaphoreType.DMA,
        ],
    )
    def k(m_hbm, oval_hbm, oidx_hbm, buf0, buf1, sval, sidx, sem0, sem1):
        cid = jax.lax.axis_index("c")
        sid = jax.lax.axis_index("s")
        wid = sid * 2 + cid
        bufs = (buf0, buf1)
        sems = (sem0, sem1)
        lane = jax.lax.iota(jnp.int32, lanes)
        neg = jnp.full((lanes,), -jnp.inf, jnp.float32)

        def startg(g, slot):
            @pl.when(g < ngroups)
            def _():
                pltpu.async_copy(m_hbm.at[pl.ds(g * lanes, lanes), :],
                                 bufs[slot], sems[slot])

        def compute(g, buf):
            for i in range(lanes):
                def body(j, carry):
                    accs, colv = carry
                    new = []
                    for w, (amax, aidx) in enumerate(accs):
                        v = buf[i, pl.ds((j * nway + w) * lanes, lanes)]
                        cw = colv + w * lanes
                        upd = v > amax
                        new.append((jnp.where(upd, v, amax),
                                    jnp.where(upd, cw, aidx)))
                    return (tuple(new), colv + nway * lanes)

                init = (tuple((neg, jnp.zeros((lanes,), jnp.int32))
                              for _ in range(nway)), lane)
                accs, colv = jax.lax.fori_loop(0, npair, body, init)
                cmax, cidx = accs[0]
                for amax, aidx in accs[1:]:
                    better = (amax > cmax) | ((amax == cmax) & (aidx < cidx))
                    cmax = jnp.where(better, amax, cmax)
                    cidx = jnp.where(better, aidx, cidx)
                for w in range(rem):
                    v = buf[i, pl.ds((npair * nway + w) * lanes, lanes)]
                    cw = colv + w * lanes
                    upd = v > cmax
                    cmax = jnp.where(upd, v, cmax)
                    cidx = jnp.where(upd, cw, cidx)
                v = buf[i, pl.ds(cols - lanes, lanes)]
                v = jnp.where(lane >= nfull * lanes - (cols - lanes), v, neg)
                upd = v > cmax
                cmax = jnp.where(upd, v, cmax)
                cidx = jnp.where(upd, cols - lanes + lane, cidx)
                sval[pl.ds(i * lanes, lanes)] = cmax
                sidx[pl.ds(i * lanes, lanes)] = cidx
            pltpu.sync_copy(sval, oval_hbm.at[g])
            pltpu.sync_copy(sidx, oidx_hbm.at[g])

        def outer(tt, _):
            for slot in (0, 1):
                g = wid + nworkers * (2 * tt + slot)

                @pl.when(g < ngroups)
                def _(g=g, slot=slot):
                    pltpu.make_async_copy(m_hbm.at[pl.ds(g * lanes, lanes), :],
                                          bufs[slot], sems[slot]).wait()
                    compute(g, bufs[slot])
                    startg(g + 2 * nworkers, slot)
            return 0

        startg(wid, 0)
        startg(wid + nworkers, 1)
        jax.lax.fori_loop(0, (max_t + 1) // 2, outer, 0)

    return k(mat)


def _tables_kernel(val_ref, idx_ref, m3_ref, m2_ref, m1_ref, m0_ref,
                   p4_ref, p3_ref, p2_ref, p1_ref, p0_ref):
    val = val_ref[...]
    idx = idx_ref[...]
    m = jnp.max(val, axis=1, keepdims=True)
    p4_ref[...] = jnp.min(jnp.where(val == m, idx, jnp.int32(2**30)),
                          axis=1, keepdims=True)
    p3_ref[...] = _first_argmax(m3_ref[...])
    p2_ref[...] = _first_argmax(m2_ref[...])
    p1_ref[...] = _first_argmax(m1_ref[...])
    p0_ref[...] = _first_argmax(m0_ref[...])


def _parent_tables(m4, m3, m2, m1, m0):
    lanes = 16
    oval, oidx = _sc_row_partials(m4)
    val = oval.reshape(CLS[5], lanes)
    idx = oidx.reshape(CLS[5], lanes)
    full = lambda shape: pl.BlockSpec(shape, lambda: (0,) * len(shape))
    return pl.pallas_call(
        _tables_kernel,
        in_specs=[
            full((CLS[5], lanes)),
            full((CLS[5], lanes)),
            full(m3.shape),
            full(m2.shape),
            full(m1.shape),
            full(m0.shape),
        ],
        out_specs=[
            full((CLS[5], 1)),
            full((CLS[4], 1)),
            full((CLS[3], 1)),
            full((CLS[2], 1)),
            full((CLS[1], 1)),
        ],
        out_shape=[
            jax.ShapeDtypeStruct((CLS[5], 1), jnp.int32),
            jax.ShapeDtypeStruct((CLS[4], 1), jnp.int32),
            jax.ShapeDtypeStruct((CLS[3], 1), jnp.int32),
            jax.ShapeDtypeStruct((CLS[2], 1), jnp.int32),
            jax.ShapeDtypeStruct((CLS[1], 1), jnp.int32),
        ],
    )(val, idx, m3, m2, m1, m0)


def _first_argmax(vals):
    m = jnp.max(vals, axis=1, keepdims=True)
    iota = jax.lax.broadcasted_iota(jnp.int32, vals.shape, 1)
    return jnp.min(jnp.where(vals == m, iota, jnp.int32(2**30)),
                   axis=1, keepdims=True)


def _head_kernel(x_ref, w_ref, b_ref, sm_ref, pred_ref):
    logits = jnp.dot(x_ref[...], w_ref[...],
                     preferred_element_type=jnp.float32) + b_ref[...]
    m = jnp.max(logits, axis=1, keepdims=True)
    e = jnp.exp(logits - m)
    s = jnp.sum(e, axis=1, keepdims=True)
    sm = e / s
    sm_ref[...] = sm
    pred_ref[...] = _first_argmax(sm)


def _chain_kernel(pred5_ref, p4_ref, p3_ref, p2_ref, p1_ref, p0_ref,
                  o4_ref, o3_ref, o2_ref, o1_ref, o0_ref):
    pred = pred5_ref[...]
    bsz = pred.shape[0]
    steps = ((p4_ref, o4_ref, CLS[5], CLS[4]),
             (p3_ref, o3_ref, CLS[4], CLS[3]),
             (p2_ref, o2_ref, CLS[3], CLS[2]),
             (p1_ref, o1_ref, CLS[2], CLS[1]),
             (p0_ref, o0_ref, CLS[1], CLS[0]))
    for t_ref, o_ref, dom, rng in steps:
        iota = jax.lax.broadcasted_iota(jnp.int32, (bsz, dom), 1)
        mask = pred == iota
        pred = jnp.sum(jnp.where(mask, t_ref[...], 0), axis=1, keepdims=True)
        iota2 = jax.lax.broadcasted_iota(jnp.int32, (bsz, rng), 1)
        o_ref[...] = (pred == iota2).astype(jnp.float32)


def kernel(x, W, b, m0, m1, m2, m3, m4):
    n = x.shape[0]
    d_in = x.shape[1]
    grid = n // BATCH_BLK

    p4, p3, p2, p1, p0 = _parent_tables(m4, m3, m2, m1, m0)
    tables = (p4.reshape(1, CLS[5]), p3.reshape(1, CLS[4]),
              p2.reshape(1, CLS[3]), p1.reshape(1, CLS[2]),
              p0.reshape(1, CLS[1]))

    sm, pred5 = pl.pallas_call(
        _head_kernel,
        grid=(grid,),
        in_specs=[
            pl.BlockSpec((BATCH_BLK, d_in), lambda i: (i, 0)),
            pl.BlockSpec((d_in, CLS[5]), lambda i: (0, 0)),
            pl.BlockSpec((1, CLS[5]), lambda i: (0, 0)),
        ],
        out_specs=[
            pl.BlockSpec((BATCH_BLK, CLS[5]), lambda i: (i, 0)),
            pl.BlockSpec((BATCH_BLK, 1), lambda i: (i, 0)),
        ],
        out_shape=[
            jax.ShapeDtypeStruct((n, CLS[5]), jnp.float32),
            jax.ShapeDtypeStruct((n, 1), jnp.int32),
        ],
        compiler_params=pltpu.CompilerParams(
            vmem_limit_bytes=100 * 1024 * 1024),
    )(x, W, b.reshape(1, CLS[5]))

    o4, o3, o2, o1, o0 = pl.pallas_call(
        _chain_kernel,
        grid=(grid,),
        in_specs=[
            pl.BlockSpec((BATCH_BLK, 1), lambda i: (i, 0)),
            pl.BlockSpec((1, CLS[5]), lambda i: (0, 0)),
            pl.BlockSpec((1, CLS[4]), lambda i: (0, 0)),
            pl.BlockSpec((1, CLS[3]), lambda i: (0, 0)),
            pl.BlockSpec((1, CLS[2]), lambda i: (0, 0)),
            pl.BlockSpec((1, CLS[1]), lambda i: (0, 0)),
        ],
        out_specs=[
            pl.BlockSpec((BATCH_BLK, CLS[4]), lambda i: (i, 0)),
            pl.BlockSpec((BATCH_BLK, CLS[3]), lambda i: (i, 0)),
            pl.BlockSpec((BATCH_BLK, CLS[2]), lambda i: (i, 0)),
            pl.BlockSpec((BATCH_BLK, CLS[1]), lambda i: (i, 0)),
            pl.BlockSpec((BATCH_BLK, CLS[0]), lambda i: (i, 0)),
        ],
        out_shape=[
            jax.ShapeDtypeStruct((n, CLS[4]), jnp.float32),
            jax.ShapeDtypeStruct((n, CLS[3]), jnp.float32),
            jax.ShapeDtypeStruct((n, CLS[2]), jnp.float32),
            jax.ShapeDtypeStruct((n, CLS[1]), jnp.float32),
            jax.ShapeDtypeStruct((n, CLS[0]), jnp.float32),
        ],
        compiler_params=pltpu.CompilerParams(
            vmem_limit_bytes=100 * 1024 * 1024),
    )(pred5, *tables)

    return (o0, o1, o2, o3, o4, sm)
